# baseline (speedup 1.0000x reference)
import functools

import jax
import jax.numpy as jnp
from jax import lax
from jax.experimental import pallas as pl
from jax.experimental.pallas import tpu as pltpu
from jax.experimental.pallas import tpu_sc as plsc

NUM_DENSE = 13
NUM_SPARSE = 26
VOCAB = 1000
EMBED = 64
BATCH = 4096
NPAIR = NUM_SPARSE // 2
XPAD = 128

NC = 2
NS = 16
NW = NC * NS

N_BCHUNK = 4
CB = BATCH // N_BCHUNK
SLAB = CB // NW
CHUNK = 2 * SLAB
N_IDX_C = CB * NUM_SPARSE
PLANE = 2 * CB


@functools.cache
def _build_sc_gather(bchunk):
    mesh = plsc.VectorSubcoreMesh(
        core_axis_name="c", subcore_axis_name="s",
        num_cores=NC, num_subcores=NS)

    @functools.partial(
        pl.kernel,
        out_type=jax.ShapeDtypeStruct((N_IDX_C, EMBED), jnp.float32),
        mesh=mesh,
        scratch_types=[
            pltpu.VMEM((SLAB, NUM_DENSE + NUM_SPARSE), jnp.float32),
            pltpu.VMEM((CHUNK,), jnp.int32),
            pltpu.VMEM((CHUNK,), jnp.int32),
            pltpu.VMEM((CHUNK, EMBED), jnp.float32),
            pltpu.VMEM((CHUNK, EMBED), jnp.float32),
            pltpu.SemaphoreType.DMA,
            pltpu.SemaphoreType.DMA,
        ],
        compiler_params=pltpu.CompilerParams(use_tc_tiling_on_sc=False,
                                             needs_layout_passes=False),
    )
    def _sc_gather(x_hbm, table_hbm, out_hbm, slab_v, cidx0, cidx1,
                   buf0, buf1, sem0, sem1):
        wid = lax.axis_index("s") * NC + lax.axis_index("c")
        row0 = bchunk * CB + wid * SLAB
        pltpu.sync_copy(x_hbm.at[pl.ds(row0, SLAB)], slab_v)

        lane = jnp.arange(16, dtype=jnp.int32)

        def fill_cidx(cidx, c):
            for kk in range(CHUNK // 16):
                k = kk * 16 + lane
                b_local = k >> 1
                p = k & 1
                col = NUM_DENSE + 2 * c + p
                vals = plsc.load_gather(slab_v, [b_local, col])
                iv = vals.astype(jnp.int32)
                iv = jnp.where(iv == -1, VOCAB - 1, iv)
                cidx[pl.ds(kk * 16, 16)] = iv + 2 * VOCAB * c + VOCAB * p

        def pair(c0, c1):
            fill_cidx(cidx0, c0)
            fill_cidx(cidx1, c1)
            cp0 = pltpu.async_copy(table_hbm.at[cidx0], buf0, sem0)
            cp1 = pltpu.async_copy(table_hbm.at[cidx1], buf1, sem1)
            cp0.wait()
            pltpu.sync_copy(buf0,
                            out_hbm.at[pl.ds(c0 * PLANE + wid * CHUNK, CHUNK)])
            cp1.wait()
            pltpu.sync_copy(buf1,
                            out_hbm.at[pl.ds(c1 * PLANE + wid * CHUNK, CHUNK)])

        def step(j, carry):
            pair(2 * j, 2 * j + 1)
            return carry

        lax.fori_loop(0, NPAIR // 2, step, 0)
        c_last = NPAIR - 1
        fill_cidx(cidx0, c_last)
        cp0 = pltpu.async_copy(table_hbm.at[cidx0], buf0, sem0)
        cp0.wait()
        pltpu.sync_copy(buf0,
                        out_hbm.at[pl.ds(c_last * PLANE + wid * CHUNK, CHUNK)])

    return _sc_gather


BLK = 256


def _mlp_body(x_ref, sf_ref, w1d_ref, w1p_ref, b1_ref, w2_ref, b2_ref,
              w3_ref, b3_ref, out_ref):
    xd = x_ref[:, :NUM_DENSE]
    h1 = jnp.dot(xd, w1d_ref[...], preferred_element_type=jnp.float32)
    for c in range(NPAIR):
        h1 = h1 + jnp.dot(sf_ref[c], w1p_ref[c],
                          preferred_element_type=jnp.float32)
    h1 = jnp.maximum(h1 + b1_ref[...], 0.0)
    h2 = jnp.dot(h1, w2_ref[...], preferred_element_type=jnp.float32)
    h2 = jnp.maximum(h2 + b2_ref[...], 0.0)
    out_ref[...] = (jnp.dot(h2, w3_ref[...], preferred_element_type=jnp.float32)
                    + b3_ref[...])


def _mlp(bchunk, x, sf_pm, w1d, w1p, b1, w2, b2, w3, b3):
    grid = (CB // BLK,)
    blk0 = bchunk * (CB // BLK)
    const2 = lambda i: (0, 0)
    const3 = lambda i: (0, 0, 0)
    return pl.pallas_call(
        _mlp_body,
        grid=grid,
        in_specs=[
            pl.BlockSpec((BLK, x.shape[1]), lambda i: (blk0 + i, 0)),
            pl.BlockSpec((NPAIR, BLK, 2 * EMBED), lambda i: (0, i, 0)),
            pl.BlockSpec(w1d.shape, const2),
            pl.BlockSpec(w1p.shape, const3),
            pl.BlockSpec(b1.shape, const2),
            pl.BlockSpec(w2.shape, const2),
            pl.BlockSpec(b2.shape, const2),
            pl.BlockSpec(w3.shape, const2),
            pl.BlockSpec(b3.shape, const2),
        ],
        out_specs=pl.BlockSpec((BLK, 2), lambda i: (i, 0)),
        out_shape=jax.ShapeDtypeStruct((CB, 2), jnp.float32),
        compiler_params=pltpu.CompilerParams(
            dimension_semantics=("parallel",),
        ),
    )(x, sf_pm, w1d, w1p, b1, w2, b2, w3, b3)


def kernel(x, codebook, W1, b1, W2, b2, W3, b3, dense_index, sparse_index):
    table = codebook.reshape(NUM_SPARSE * VOCAB, EMBED)

    w1d = W1[:NUM_DENSE]
    w1p = W1[NUM_DENSE:].reshape(NPAIR, 2 * EMBED, 512)
    b1r, b2r, b3r = b1.reshape(1, -1), b2.reshape(1, -1), b3.reshape(1, -1)

    outs = []
    for k in range(N_BCHUNK):
        sf_pm = _build_sc_gather(k)(x, table).reshape(NPAIR, CB, 2 * EMBED)
        outs.append(_mlp(k, x, sf_pm, w1d, w1p, b1r, W2, b2r, W3, b3r))
    return jnp.concatenate(outs, axis=0)


# speedup vs baseline: 1.1383x; 1.1383x over previous
import functools

import jax
import jax.numpy as jnp
from jax import lax
from jax.experimental import pallas as pl
from jax.experimental.pallas import tpu as pltpu
from jax.experimental.pallas import tpu_sc as plsc

NUM_DENSE = 13
NUM_SPARSE = 26
VOCAB = 1000
EMBED = 64
BATCH = 4096
NPAIR = NUM_SPARSE // 2
XPAD = 128

NC = 2
NS = 16
NW = NC * NS

N_BCHUNK = 2
CB = BATCH // N_BCHUNK
SLAB = CB // NW
CHUNK = 2 * SLAB
N_IDX_C = CB * NUM_SPARSE
PLANE = 2 * CB


@functools.cache
def _build_sc_gather(bchunk):
    mesh = plsc.VectorSubcoreMesh(
        core_axis_name="c", subcore_axis_name="s",
        num_cores=NC, num_subcores=NS)

    @functools.partial(
        pl.kernel,
        out_type=jax.ShapeDtypeStruct((N_IDX_C, EMBED), jnp.float32),
        mesh=mesh,
        scratch_types=[
            pltpu.VMEM((SLAB, NUM_DENSE + NUM_SPARSE), jnp.float32),
            pltpu.VMEM((CHUNK,), jnp.int32),
            pltpu.VMEM((CHUNK,), jnp.int32),
            pltpu.VMEM((CHUNK, EMBED), jnp.float32),
            pltpu.VMEM((CHUNK, EMBED), jnp.float32),
            pltpu.SemaphoreType.DMA,
            pltpu.SemaphoreType.DMA,
        ],
        compiler_params=pltpu.CompilerParams(use_tc_tiling_on_sc=False,
                                             needs_layout_passes=False),
    )
    def _sc_gather(x_hbm, table_hbm, out_hbm, slab_v, cidx0, cidx1,
                   buf0, buf1, sem0, sem1):
        wid = lax.axis_index("s") * NC + lax.axis_index("c")
        row0 = bchunk * CB + wid * SLAB
        pltpu.sync_copy(x_hbm.at[pl.ds(row0, SLAB)], slab_v)

        lane = jnp.arange(16, dtype=jnp.int32)

        def fill_cidx(cidx, c):
            for kk in range(CHUNK // 16):
                k = kk * 16 + lane
                b_local = k >> 1
                p = k & 1
                col = NUM_DENSE + 2 * c + p
                vals = plsc.load_gather(slab_v, [b_local, col])
                iv = vals.astype(jnp.int32)
                iv = jnp.where(iv == -1, VOCAB - 1, iv)
                cidx[pl.ds(kk * 16, 16)] = iv + 2 * VOCAB * c + VOCAB * p

        def pair(c0, c1):
            fill_cidx(cidx0, c0)
            fill_cidx(cidx1, c1)
            cp0 = pltpu.async_copy(table_hbm.at[cidx0], buf0, sem0)
            cp1 = pltpu.async_copy(table_hbm.at[cidx1], buf1, sem1)
            cp0.wait()
            pltpu.sync_copy(buf0,
                            out_hbm.at[pl.ds(c0 * PLANE + wid * CHUNK, CHUNK)])
            cp1.wait()
            pltpu.sync_copy(buf1,
                            out_hbm.at[pl.ds(c1 * PLANE + wid * CHUNK, CHUNK)])

        def step(j, carry):
            pair(2 * j, 2 * j + 1)
            return carry

        lax.fori_loop(0, NPAIR // 2, step, 0)
        c_last = NPAIR - 1
        fill_cidx(cidx0, c_last)
        cp0 = pltpu.async_copy(table_hbm.at[cidx0], buf0, sem0)
        cp0.wait()
        pltpu.sync_copy(buf0,
                        out_hbm.at[pl.ds(c_last * PLANE + wid * CHUNK, CHUNK)])

    return _sc_gather


BLK = 256


def _mlp_body(x_ref, sf_ref, w1d_ref, w1p_ref, b1_ref, w2_ref, b2_ref,
              w3_ref, b3_ref, out_ref):
    xd = x_ref[:, :NUM_DENSE]
    h1 = jnp.dot(xd, w1d_ref[...], preferred_element_type=jnp.float32)
    for c in range(NPAIR):
        h1 = h1 + jnp.dot(sf_ref[c], w1p_ref[c],
                          preferred_element_type=jnp.float32)
    h1 = jnp.maximum(h1 + b1_ref[...], 0.0)
    h2 = jnp.dot(h1, w2_ref[...], preferred_element_type=jnp.float32)
    h2 = jnp.maximum(h2 + b2_ref[...], 0.0)
    out_ref[...] = (jnp.dot(h2, w3_ref[...], preferred_element_type=jnp.float32)
                    + b3_ref[...])


def _mlp(bchunk, x, sf_pm, w1d, w1p, b1, w2, b2, w3, b3):
    grid = (CB // BLK,)
    blk0 = bchunk * (CB // BLK)
    const2 = lambda i: (0, 0)
    const3 = lambda i: (0, 0, 0)
    return pl.pallas_call(
        _mlp_body,
        grid=grid,
        in_specs=[
            pl.BlockSpec((BLK, x.shape[1]), lambda i: (blk0 + i, 0)),
            pl.BlockSpec((NPAIR, BLK, 2 * EMBED), lambda i: (0, i, 0)),
            pl.BlockSpec(w1d.shape, const2),
            pl.BlockSpec(w1p.shape, const3),
            pl.BlockSpec(b1.shape, const2),
            pl.BlockSpec(w2.shape, const2),
            pl.BlockSpec(b2.shape, const2),
            pl.BlockSpec(w3.shape, const2),
            pl.BlockSpec(b3.shape, const2),
        ],
        out_specs=pl.BlockSpec((BLK, 2), lambda i: (i, 0)),
        out_shape=jax.ShapeDtypeStruct((CB, 2), jnp.float32),
        compiler_params=pltpu.CompilerParams(
            dimension_semantics=("parallel",),
        ),
    )(x, sf_pm, w1d, w1p, b1, w2, b2, w3, b3)


def kernel(x, codebook, W1, b1, W2, b2, W3, b3, dense_index, sparse_index):
    table = codebook.reshape(NUM_SPARSE * VOCAB, EMBED)

    w1d = W1[:NUM_DENSE]
    w1p = W1[NUM_DENSE:].reshape(NPAIR, 2 * EMBED, 512)
    b1r, b2r, b3r = b1.reshape(1, -1), b2.reshape(1, -1), b3.reshape(1, -1)

    outs = []
    for k in range(N_BCHUNK):
        sf_pm = _build_sc_gather(k)(x, table).reshape(NPAIR, CB, 2 * EMBED)
        outs.append(_mlp(k, x, sf_pm, w1d, w1p, b1r, W2, b2r, W3, b3r))
    return jnp.concatenate(outs, axis=0)


# speedup vs baseline: 1.1577x; 1.0170x over previous
import functools

import jax
import jax.numpy as jnp
from jax import lax
from jax.experimental import pallas as pl
from jax.experimental.pallas import tpu as pltpu
from jax.experimental.pallas import tpu_sc as plsc

NUM_DENSE = 13
NUM_SPARSE = 26
VOCAB = 1000
EMBED = 64
BATCH = 4096
NPAIR = NUM_SPARSE // 2
XPAD = 128

NC = 2
NS = 16
NW = NC * NS

N_BCHUNK = 2
CB = BATCH // N_BCHUNK
SLAB = CB // NW
CHUNK = 2 * SLAB
N_IDX_C = CB * NUM_SPARSE
PLANE = 2 * CB


@functools.cache
def _build_sc_gather(bchunk):
    mesh = plsc.VectorSubcoreMesh(
        core_axis_name="c", subcore_axis_name="s",
        num_cores=NC, num_subcores=NS)

    @functools.partial(
        pl.kernel,
        out_type=jax.ShapeDtypeStruct((N_IDX_C, EMBED), jnp.float32),
        mesh=mesh,
        scratch_types=[
            pltpu.VMEM((SLAB, NUM_DENSE + NUM_SPARSE), jnp.float32),
            [pltpu.VMEM((CHUNK,), jnp.int32) for _ in range(4)],
            [pltpu.VMEM((CHUNK, EMBED), jnp.float32) for _ in range(4)],
            [pltpu.SemaphoreType.DMA for _ in range(4)],
            [pltpu.SemaphoreType.DMA for _ in range(4)],
        ],
        compiler_params=pltpu.CompilerParams(use_tc_tiling_on_sc=False,
                                             needs_layout_passes=False),
    )
    def _sc_gather(x_hbm, table_hbm, out_hbm, slab_v, cidx, buf, gsem, wsem):
        wid = lax.axis_index("s") * NC + lax.axis_index("c")
        row0 = bchunk * CB + wid * SLAB
        pltpu.sync_copy(x_hbm.at[pl.ds(row0, SLAB)], slab_v)

        lane = jnp.arange(16, dtype=jnp.int32)

        def fill_cidx(s, c):
            for kk in range(CHUNK // 16):
                k = kk * 16 + lane
                b_local = k >> 1
                p = k & 1
                col = NUM_DENSE + 2 * c + p
                vals = plsc.load_gather(slab_v, [b_local, col])
                iv = vals.astype(jnp.int32)
                iv = jnp.where(iv == -1, VOCAB - 1, iv)
                cidx[s][pl.ds(kk * 16, 16)] = iv + 2 * VOCAB * c + VOCAB * p

        def gather(s, c):
            return pltpu.async_copy(table_hbm.at[cidx[s]], buf[s], gsem[s])

        def write(s, c):
            dst = out_hbm.at[pl.ds(c * PLANE + wid * CHUNK, CHUNK)]
            return pltpu.async_copy(buf[s], dst, wsem[s])

        gcp = []
        for s in range(4):
            fill_cidx(s, s)
            gcp.append(gather(s, s))
        wcp = [None] * 4
        for q in range(2):
            for s in range(4):
                gcp[s].wait()
                wcp[s] = write(s, 4 * q + s)
            for s in range(4):
                fill_cidx(s, 4 * q + 4 + s)
                wcp[s].wait()
                gcp[s] = gather(s, 4 * q + 4 + s)
        for s in range(4):
            gcp[s].wait()
            wcp[s] = write(s, 8 + s)
        fill_cidx(0, 12)
        wcp[0].wait()
        g = gather(0, 12)
        g.wait()
        w = write(0, 12)
        for s in range(1, 4):
            wcp[s].wait()
        w.wait()

    return _sc_gather


BLK = 256


def _mlp_body(x_ref, sf_ref, w1d_ref, w1p_ref, b1_ref, w2_ref, b2_ref,
              w3_ref, b3_ref, out_ref):
    xd = x_ref[:, :NUM_DENSE]
    h1 = jnp.dot(xd, w1d_ref[...], preferred_element_type=jnp.float32)
    for c in range(NPAIR):
        h1 = h1 + jnp.dot(sf_ref[c], w1p_ref[c],
                          preferred_element_type=jnp.float32)
    h1 = jnp.maximum(h1 + b1_ref[...], 0.0)
    h2 = jnp.dot(h1, w2_ref[...], preferred_element_type=jnp.float32)
    h2 = jnp.maximum(h2 + b2_ref[...], 0.0)
    out_ref[...] = (jnp.dot(h2, w3_ref[...], preferred_element_type=jnp.float32)
                    + b3_ref[...])


def _mlp(bchunk, x, sf_pm, w1d, w1p, b1, w2, b2, w3, b3):
    grid = (CB // BLK,)
    blk0 = bchunk * (CB // BLK)
    const2 = lambda i: (0, 0)
    const3 = lambda i: (0, 0, 0)
    return pl.pallas_call(
        _mlp_body,
        grid=grid,
        in_specs=[
            pl.BlockSpec((BLK, x.shape[1]), lambda i: (blk0 + i, 0)),
            pl.BlockSpec((NPAIR, BLK, 2 * EMBED), lambda i: (0, i, 0)),
            pl.BlockSpec(w1d.shape, const2),
            pl.BlockSpec(w1p.shape, const3),
            pl.BlockSpec(b1.shape, const2),
            pl.BlockSpec(w2.shape, const2),
            pl.BlockSpec(b2.shape, const2),
            pl.BlockSpec(w3.shape, const2),
            pl.BlockSpec(b3.shape, const2),
        ],
        out_specs=pl.BlockSpec((BLK, 2), lambda i: (i, 0)),
        out_shape=jax.ShapeDtypeStruct((CB, 2), jnp.float32),
        compiler_params=pltpu.CompilerParams(
            dimension_semantics=("parallel",),
        ),
    )(x, sf_pm, w1d, w1p, b1, w2, b2, w3, b3)


def kernel(x, codebook, W1, b1, W2, b2, W3, b3, dense_index, sparse_index):
    table = codebook.reshape(NUM_SPARSE * VOCAB, EMBED)

    w1d = W1[:NUM_DENSE]
    w1p = W1[NUM_DENSE:].reshape(NPAIR, 2 * EMBED, 512)
    b1r, b2r, b3r = b1.reshape(1, -1), b2.reshape(1, -1), b3.reshape(1, -1)

    outs = []
    for k in range(N_BCHUNK):
        sf_pm = _build_sc_gather(k)(x, table).reshape(NPAIR, CB, 2 * EMBED)
        outs.append(_mlp(k, x, sf_pm, w1d, w1p, b1r, W2, b2r, W3, b3r))
    return jnp.concatenate(outs, axis=0)


# speedup vs baseline: 1.1761x; 1.0159x over previous
import functools

import jax
import jax.numpy as jnp
from jax import lax
from jax.experimental import pallas as pl
from jax.experimental.pallas import tpu as pltpu
from jax.experimental.pallas import tpu_sc as plsc

NUM_DENSE = 13
NUM_SPARSE = 26
VOCAB = 1000
EMBED = 64
BATCH = 4096
NPAIR = NUM_SPARSE // 2
NFEAT = NUM_DENSE + NUM_SPARSE

NC = 2
NS = 16
NW = NC * NS

N_BCHUNK = 2
CB = BATCH // N_BCHUNK
SLAB = CB // NW
CHUNK = 2 * SLAB
N_IDX_C = CB * NUM_SPARSE
PLANE = 2 * CB


@functools.cache
def _build_sc_gather(bchunk):
    mesh = plsc.VectorSubcoreMesh(
        core_axis_name="c", subcore_axis_name="s",
        num_cores=NC, num_subcores=NS)

    @functools.partial(
        pl.kernel,
        out_type=jax.ShapeDtypeStruct((N_IDX_C, EMBED), jnp.float32),
        mesh=mesh,
        scratch_types=[
            pltpu.VMEM((SLAB * NFEAT,), jnp.float32),
            [pltpu.VMEM((CHUNK,), jnp.int32) for _ in range(4)],
            [pltpu.VMEM((CHUNK, EMBED), jnp.float32) for _ in range(4)],
            [pltpu.SemaphoreType.DMA for _ in range(4)],
            [pltpu.SemaphoreType.DMA for _ in range(4)],
        ],
        compiler_params=pltpu.CompilerParams(use_tc_tiling_on_sc=False,
                                             needs_layout_passes=False),
    )
    def _sc_gather(x_hbm, table_hbm, out_hbm, slab_v, cidx, buf, gsem, wsem):
        wid = lax.axis_index("s") * NC + lax.axis_index("c")
        row0 = bchunk * CB + wid * SLAB
        pltpu.sync_copy(x_hbm.at[pl.ds(row0 * NFEAT, SLAB * NFEAT)], slab_v)

        lane = jnp.arange(16, dtype=jnp.int32)

        def fill_cidx(s, c):
            for kk in range(CHUNK // 16):
                k = kk * 16 + lane
                b_local = k >> 1
                p = k & 1
                col = NUM_DENSE + 2 * c + p
                vals = plsc.load_gather(slab_v, [b_local * NFEAT + col])
                iv = vals.astype(jnp.int32)
                iv = jnp.where(iv == -1, VOCAB - 1, iv)
                cidx[s][pl.ds(kk * 16, 16)] = iv + 2 * VOCAB * c + VOCAB * p

        def gather(s, c):
            return pltpu.async_copy(table_hbm.at[cidx[s]], buf[s], gsem[s])

        def write(s, c):
            dst = out_hbm.at[pl.ds(c * PLANE + wid * CHUNK, CHUNK)]
            return pltpu.async_copy(buf[s], dst, wsem[s])

        gcp = []
        for s in range(4):
            fill_cidx(s, s)
            gcp.append(gather(s, s))
        wcp = [None] * 4
        for q in range(2):
            for s in range(4):
                gcp[s].wait()
                wcp[s] = write(s, 4 * q + s)
            for s in range(4):
                fill_cidx(s, 4 * q + 4 + s)
                wcp[s].wait()
                gcp[s] = gather(s, 4 * q + 4 + s)
        for s in range(4):
            gcp[s].wait()
            wcp[s] = write(s, 8 + s)
        fill_cidx(0, 12)
        wcp[0].wait()
        g = gather(0, 12)
        g.wait()
        w = write(0, 12)
        for s in range(1, 4):
            wcp[s].wait()
        w.wait()

    return _sc_gather


BLK = 512


def _mlp_body(x_ref, sf_ref, w1d_ref, w1p_ref, b1_ref, w2_ref, b2_ref,
              w3_ref, b3_ref, out_ref):
    xd = x_ref[:, :NUM_DENSE]
    h1 = jnp.dot(xd, w1d_ref[...], preferred_element_type=jnp.float32)
    for c in range(NPAIR):
        h1 = h1 + jnp.dot(sf_ref[c], w1p_ref[c],
                          preferred_element_type=jnp.float32)
    h1 = jnp.maximum(h1 + b1_ref[...], 0.0)
    h2 = jnp.dot(h1, w2_ref[...], preferred_element_type=jnp.float32)
    h2 = jnp.maximum(h2 + b2_ref[...], 0.0)
    out_ref[...] = (jnp.dot(h2, w3_ref[...], preferred_element_type=jnp.float32)
                    + b3_ref[...])


def _mlp(bchunk, x, sf_pm, w1d, w1p, b1, w2, b2, w3, b3):
    grid = (CB // BLK,)
    blk0 = bchunk * (CB // BLK)
    const2 = lambda i: (0, 0)
    const3 = lambda i: (0, 0, 0)
    return pl.pallas_call(
        _mlp_body,
        grid=grid,
        in_specs=[
            pl.BlockSpec((BLK, x.shape[1]), lambda i: (blk0 + i, 0)),
            pl.BlockSpec((NPAIR, BLK, 2 * EMBED), lambda i: (0, i, 0)),
            pl.BlockSpec(w1d.shape, const2),
            pl.BlockSpec(w1p.shape, const3),
            pl.BlockSpec(b1.shape, const2),
            pl.BlockSpec(w2.shape, const2),
            pl.BlockSpec(b2.shape, const2),
            pl.BlockSpec(w3.shape, const2),
            pl.BlockSpec(b3.shape, const2),
        ],
        out_specs=pl.BlockSpec((BLK, 2), lambda i: (i, 0)),
        out_shape=jax.ShapeDtypeStruct((CB, 2), jnp.float32),
        compiler_params=pltpu.CompilerParams(
            dimension_semantics=("parallel",),
        ),
    )(x, sf_pm, w1d, w1p, b1, w2, b2, w3, b3)


def kernel(x, codebook, W1, b1, W2, b2, W3, b3, dense_index, sparse_index):
    table = codebook.reshape(NUM_SPARSE * VOCAB, EMBED)
    xflat = x.reshape(-1)

    w1d = W1[:NUM_DENSE]
    w1p = W1[NUM_DENSE:].reshape(NPAIR, 2 * EMBED, 512)
    b1r, b2r, b3r = b1.reshape(1, -1), b2.reshape(1, -1), b3.reshape(1, -1)

    outs = []
    for k in range(N_BCHUNK):
        sf_pm = _build_sc_gather(k)(xflat, table).reshape(NPAIR, CB, 2 * EMBED)
        outs.append(_mlp(k, x, sf_pm, w1d, w1p, b1r, W2, b2r, W3, b3r))
    return jnp.concatenate(outs, axis=0)


# speedup vs baseline: 1.1951x; 1.0161x over previous
import functools

import jax
import jax.numpy as jnp
from jax import lax
from jax.experimental import pallas as pl
from jax.experimental.pallas import tpu as pltpu
from jax.experimental.pallas import tpu_sc as plsc

NUM_DENSE = 13
NUM_SPARSE = 26
VOCAB = 1000
EMBED = 64
BATCH = 4096
NPAIR = NUM_SPARSE // 2
NFEAT = NUM_DENSE + NUM_SPARSE

NC = 2
NS = 16
NW = NC * NS

N_BCHUNK = 2
CB = BATCH // N_BCHUNK
SLAB = CB // NW
CHUNK = 2 * SLAB
N_IDX_C = CB * NUM_SPARSE
PLANE = 2 * CB


@functools.cache
def _build_sc_gather(bchunk):
    mesh = plsc.VectorSubcoreMesh(
        core_axis_name="c", subcore_axis_name="s",
        num_cores=NC, num_subcores=NS)

    @functools.partial(
        pl.kernel,
        out_type=jax.ShapeDtypeStruct((N_IDX_C, EMBED), jnp.float32),
        mesh=mesh,
        scratch_types=[
            pltpu.VMEM((SLAB * NFEAT,), jnp.float32),
            [pltpu.VMEM((CHUNK,), jnp.int32) for _ in range(4)],
            [pltpu.VMEM((CHUNK, EMBED), jnp.float32) for _ in range(4)],
            [pltpu.SemaphoreType.DMA for _ in range(4)],
            [pltpu.SemaphoreType.DMA for _ in range(4)],
        ],
        compiler_params=pltpu.CompilerParams(use_tc_tiling_on_sc=False,
                                             needs_layout_passes=False),
    )
    def _sc_gather(x_hbm, table_hbm, out_hbm, slab_v, cidx, buf, gsem, wsem):
        wid = lax.axis_index("s") * NC + lax.axis_index("c")
        row0 = bchunk * CB + wid * SLAB
        pltpu.sync_copy(x_hbm.at[pl.ds(row0 * NFEAT, SLAB * NFEAT)], slab_v)

        lane = jnp.arange(16, dtype=jnp.int32)

        def fill_cidx(s, c):
            for kk in range(CHUNK // 16):
                k = kk * 16 + lane
                b_local = k >> 1
                p = k & 1
                col = NUM_DENSE + 2 * c + p
                vals = plsc.load_gather(slab_v, [b_local * NFEAT + col])
                iv = vals.astype(jnp.int32)
                iv = jnp.where(iv == -1, VOCAB - 1, iv)
                cidx[s][pl.ds(kk * 16, 16)] = iv + 2 * VOCAB * c + VOCAB * p

        def gather(s, c):
            return pltpu.async_copy(table_hbm.at[cidx[s]], buf[s], gsem[s])

        def write(s, c):
            dst = out_hbm.at[pl.ds(c * PLANE + wid * CHUNK, CHUNK)]
            return pltpu.async_copy(buf[s], dst, wsem[s])

        gcp = []
        for s in range(4):
            fill_cidx(s, s)
            gcp.append(gather(s, s))
        wcp = [None] * 4
        for q in range(2):
            for s in range(4):
                gcp[s].wait()
                wcp[s] = write(s, 4 * q + s)
            for s in range(4):
                fill_cidx(s, 4 * q + 4 + s)
                wcp[s].wait()
                gcp[s] = gather(s, 4 * q + 4 + s)
        for s in range(4):
            gcp[s].wait()
            wcp[s] = write(s, 8 + s)
        fill_cidx(0, 12)
        wcp[0].wait()
        g = gather(0, 12)
        g.wait()
        w = write(0, 12)
        for s in range(1, 4):
            wcp[s].wait()
        w.wait()

    return _sc_gather


BLK = 512


def _mlp_body(x_ref, sf_ref, w1d_ref, w1p_ref, b1_ref, w2_ref, b2_ref,
              w3_ref, b3_ref, out_ref):
    xd = x_ref[:, :NUM_DENSE]
    h1 = jnp.dot(xd, w1d_ref[...], preferred_element_type=jnp.float32)
    for c in range(NPAIR):
        h1 = h1 + jnp.dot(sf_ref[c].astype(jnp.bfloat16), w1p_ref[c],
                          preferred_element_type=jnp.float32)
    h1 = jnp.maximum(h1 + b1_ref[...], 0.0)
    h2 = jnp.dot(h1.astype(jnp.bfloat16), w2_ref[...],
                 preferred_element_type=jnp.float32)
    h2 = jnp.maximum(h2 + b2_ref[...], 0.0)
    out_ref[...] = (jnp.dot(h2, w3_ref[...], preferred_element_type=jnp.float32)
                    + b3_ref[...])


def _mlp(bchunk, x, sf_pm, w1d, w1p, b1, w2, b2, w3, b3):
    grid = (CB // BLK,)
    blk0 = bchunk * (CB // BLK)
    const2 = lambda i: (0, 0)
    const3 = lambda i: (0, 0, 0)
    return pl.pallas_call(
        _mlp_body,
        grid=grid,
        in_specs=[
            pl.BlockSpec((BLK, x.shape[1]), lambda i: (blk0 + i, 0)),
            pl.BlockSpec((NPAIR, BLK, 2 * EMBED), lambda i: (0, i, 0)),
            pl.BlockSpec(w1d.shape, const2),
            pl.BlockSpec(w1p.shape, const3),
            pl.BlockSpec(b1.shape, const2),
            pl.BlockSpec(w2.shape, const2),
            pl.BlockSpec(b2.shape, const2),
            pl.BlockSpec(w3.shape, const2),
            pl.BlockSpec(b3.shape, const2),
        ],
        out_specs=pl.BlockSpec((BLK, 2), lambda i: (i, 0)),
        out_shape=jax.ShapeDtypeStruct((CB, 2), jnp.float32),
        compiler_params=pltpu.CompilerParams(
            dimension_semantics=("parallel",),
        ),
    )(x, sf_pm, w1d, w1p, b1, w2, b2, w3, b3)


def kernel(x, codebook, W1, b1, W2, b2, W3, b3, dense_index, sparse_index):
    table = codebook.reshape(NUM_SPARSE * VOCAB, EMBED)
    xflat = x.reshape(-1)

    w1d = W1[:NUM_DENSE]
    w1p = W1[NUM_DENSE:].reshape(NPAIR, 2 * EMBED, 512).astype(jnp.bfloat16)
    W2 = W2.astype(jnp.bfloat16)
    b1r, b2r, b3r = b1.reshape(1, -1), b2.reshape(1, -1), b3.reshape(1, -1)

    outs = []
    for k in range(N_BCHUNK):
        sf_pm = _build_sc_gather(k)(xflat, table).reshape(NPAIR, CB, 2 * EMBED)
        outs.append(_mlp(k, x, sf_pm, w1d, w1p, b1r, W2, b2r, W3, b3r))
    return jnp.concatenate(outs, axis=0)
